# Initial kernel scaffold; baseline (speedup 1.0000x reference)
import jax
import jax.numpy as jnp
from jax.experimental import pallas as pl


def kernel(source, relations, entity_embeddings, relation_embeddings):
    raise NotImplementedError("write your pallas kernel here")


# speedup vs baseline: 1.7807x; 1.7807x over previous
import functools

import jax
import jax.numpy as jnp
from jax import lax
from jax.experimental import pallas as pl
from jax.experimental.pallas import tpu as pltpu
from jax.experimental.pallas import tpu_sc as plsc

B = 16384
D = 128
L = 16
NC = 2
NS = 16
NW = NC * NS
BPW = B // NW
C = 128
NCHUNK = BPW // C


def _hsum16(v):
    dnums = lax.GatherDimensionNumbers(
        offset_dims=(), collapsed_slice_dims=(0,), start_index_map=(0,))
    for k in (1, 2, 4, 8):
        perm = lax.iota(jnp.int32, L) ^ k
        v = v + lax.gather(v, perm[:, None], dnums, slice_sizes=(1,),
                           mode=lax.GatherScatterMode.PROMISE_IN_BOUNDS)
    return v


def _rsqrt16(x):
    xi = lax.bitcast_convert_type(x, jnp.int32)
    yi = jnp.int32(0x5F3759DF) - lax.shift_right_logical(xi, 1)
    y = lax.bitcast_convert_type(yi, jnp.float32)
    for _ in range(3):
        y = y * (1.5 - 0.5 * x * y * y)
    return y


def _sc_body(src, rel, ent, reltab, out, idx_e, idx_r, rows_e, rows_r, sem):
    wid = lax.axis_index("s") * NC + lax.axis_index("c")
    base = wid * BPW

    def chunk(c, carry):
        cb = base + c * C
        pltpu.sync_copy(src.at[pl.ds(cb, C)], idx_e)
        pltpu.sync_copy(rel.at[pl.ds(cb, C)], idx_r)
        cp1 = pltpu.async_copy(ent.at[idx_e], rows_e, sem)
        cp2 = pltpu.async_copy(reltab.at[idx_r], rows_r, sem)
        cp1.wait()
        cp2.wait()

        def row(r, carry2):
            acc_e = jnp.zeros((L,), jnp.float32)
            acc_r = jnp.zeros((L,), jnp.float32)
            for j in range(D // L):
                ve = rows_e[r, pl.ds(j * L, L)]
                vr = rows_r[r, pl.ds(j * L, L)]
                acc_e = acc_e + ve * ve
                acc_r = acc_r + vr * vr
            se = jnp.maximum(_hsum16(acc_e), jnp.float32(1e-12))
            sr = jnp.maximum(_hsum16(acc_r), jnp.float32(1e-12))
            inv_e = _rsqrt16(se)
            inv_r = _rsqrt16(sr)
            for j in range(D // L):
                ve = rows_e[r, pl.ds(j * L, L)]
                vr = rows_r[r, pl.ds(j * L, L)]
                rows_e[r, pl.ds(j * L, L)] = ve * inv_e + vr * inv_r
            return carry2

        lax.fori_loop(0, C, row, 0)
        pltpu.sync_copy(rows_e, out.at[pl.ds(cb, C)])
        return carry

    lax.fori_loop(0, NCHUNK, chunk, 0)


@jax.jit
def kernel(source, relations, entity_embeddings, relation_embeddings):
    src = source.astype(jnp.int32)
    rel = relations.astype(jnp.int32)
    mesh = plsc.VectorSubcoreMesh(core_axis_name="c", subcore_axis_name="s")
    k = functools.partial(
        pl.kernel,
        out_type=jax.ShapeDtypeStruct((B, D), jnp.float32),
        mesh=mesh,
        scratch_types=[
            pltpu.VMEM((C,), jnp.int32),
            pltpu.VMEM((C,), jnp.int32),
            pltpu.VMEM((C, D), jnp.float32),
            pltpu.VMEM((C, D), jnp.float32),
            pltpu.SemaphoreType.DMA,
        ],
    )(_sc_body)
    return k(src, rel, entity_embeddings, relation_embeddings)


# speedup vs baseline: 2.0001x; 1.1232x over previous
import functools

import jax
import jax.numpy as jnp
from jax import lax
from jax.experimental import pallas as pl
from jax.experimental.pallas import tpu as pltpu
from jax.experimental.pallas import tpu_sc as plsc

B = 16384
D = 128
L = 16
NC = 2
NS = 16
NW = NC * NS
BPW = B // NW
C = 128
NCHUNK = BPW // C


def _hsum16(v):
    dnums = lax.GatherDimensionNumbers(
        offset_dims=(), collapsed_slice_dims=(0,), start_index_map=(0,))
    for k in (1, 2, 4, 8):
        perm = lax.iota(jnp.int32, L) ^ k
        v = v + lax.gather(v, perm[:, None], dnums, slice_sizes=(1,),
                           mode=lax.GatherScatterMode.PROMISE_IN_BOUNDS)
    return v


def _rsqrt16(x):
    xi = lax.bitcast_convert_type(x, jnp.int32)
    yi = jnp.int32(0x5F3759DF) - lax.shift_right_logical(xi, 1)
    y = lax.bitcast_convert_type(yi, jnp.float32)
    for _ in range(3):
        y = y * (1.5 - 0.5 * x * y * y)
    return y


def _sc_body(src, rel, ent, reltab, out, idx_e, idx_r, rows_e, rows_r,
             sem_in0, sem_in1, sem_out0, sem_out1):
    wid = lax.axis_index("s") * NC + lax.axis_index("c")
    base = wid * BPW
    sem_in = (sem_in0, sem_in1)
    sem_out = (sem_out0, sem_out1)

    def start(c):
        s = c % 2
        cb = base + c * C
        pltpu.sync_copy(src.at[pl.ds(cb, C)], idx_e.at[s])
        pltpu.sync_copy(rel.at[pl.ds(cb, C)], idx_r.at[s])
        pltpu.async_copy(ent.at[idx_e.at[s]], rows_e.at[s], sem_in[s])
        pltpu.async_copy(reltab.at[idx_r.at[s]], rows_r.at[s], sem_in[s])

    def wait_in(c):
        s = c % 2
        pltpu.make_async_copy(ent.at[idx_e.at[s]], rows_e.at[s], sem_in[s]).wait()
        pltpu.make_async_copy(reltab.at[idx_r.at[s]], rows_r.at[s], sem_in[s]).wait()

    def compute(c):
        s = c % 2
        re = rows_e.at[s]
        rr = rows_r.at[s]

        def row(r, carry2):
            acc_e = jnp.zeros((L,), jnp.float32)
            acc_r = jnp.zeros((L,), jnp.float32)
            for j in range(D // L):
                ve = re[r, pl.ds(j * L, L)]
                vr = rr[r, pl.ds(j * L, L)]
                acc_e = acc_e + ve * ve
                acc_r = acc_r + vr * vr
            se = jnp.maximum(_hsum16(acc_e), jnp.float32(1e-12))
            sr = jnp.maximum(_hsum16(acc_r), jnp.float32(1e-12))
            inv_e = _rsqrt16(se)
            inv_r = _rsqrt16(sr)
            for j in range(D // L):
                ve = re[r, pl.ds(j * L, L)]
                vr = rr[r, pl.ds(j * L, L)]
                re[r, pl.ds(j * L, L)] = ve * inv_e + vr * inv_r
            return carry2

        lax.fori_loop(0, C, row, 0)

    def start_out(c):
        s = c % 2
        cb = base + c * C
        pltpu.async_copy(rows_e.at[s], out.at[pl.ds(cb, C)], sem_out[s])

    def wait_out(c):
        s = c % 2
        cb = base + c * C
        pltpu.make_async_copy(rows_e.at[s], out.at[pl.ds(cb, C)], sem_out[s]).wait()

    start(0)
    for c in range(NCHUNK):
        if c + 1 < NCHUNK:
            if c >= 1:
                wait_out(c - 1)
            start(c + 1)
        wait_in(c)
        compute(c)
        start_out(c)
    wait_out(NCHUNK - 1)
    wait_out(NCHUNK - 2)


@jax.jit
def kernel(source, relations, entity_embeddings, relation_embeddings):
    src = source.astype(jnp.int32)
    rel = relations.astype(jnp.int32)
    mesh = plsc.VectorSubcoreMesh(core_axis_name="c", subcore_axis_name="s")
    k = functools.partial(
        pl.kernel,
        out_type=jax.ShapeDtypeStruct((B, D), jnp.float32),
        mesh=mesh,
        scratch_types=[
            pltpu.VMEM((2, C), jnp.int32),
            pltpu.VMEM((2, C), jnp.int32),
            pltpu.VMEM((2, C, D), jnp.float32),
            pltpu.VMEM((2, C, D), jnp.float32),
            pltpu.SemaphoreType.DMA,
            pltpu.SemaphoreType.DMA,
            pltpu.SemaphoreType.DMA,
            pltpu.SemaphoreType.DMA,
        ],
    )(_sc_body)
    return k(src, rel, entity_embeddings, relation_embeddings)
